# baseline (speedup 1.0000x reference)
import functools

import jax
import jax.numpy as jnp
from jax import lax
from jax.experimental import pallas as pl
from jax.experimental.pallas import tpu as pltpu
from jax.experimental.pallas import tpu_sc as plsc

N_NODES = 100000
N_EDGES = 200000
D = 128
R = 3
NB = 2

NC = 2
NS = 16
NW = NC * NS

EPW = 6400
EPAD = NW * EPW
G = 128
GPW = EPW // G
WV = 256
NWAVE = EPW // WV

CH = 7168
SPS = CH // NS
P = 14
NP = P * CH

ZR = 112


def _sc_body(src_hbm, dst_hbm, x_hbm, z128_hbm, z8_hbm, onestab_hbm,
             agg_hbm, deg_hbm,
             src_v, dst_v, off_v, rows_v, gsrc_v, didx_v, dones_v,
             zrows_v, zdeg_v, acc_s, dega_s, gsem, g2sem):
    c = lax.axis_index("c")
    s = lax.axis_index("s")
    wid = s * NC + c
    iota16 = lax.iota(jnp.int32, 16)

    pltpu.sync_copy(z128_hbm, zrows_v)
    pltpu.sync_copy(z8_hbm, zdeg_v)

    for r in range(R):
        pltpu.sync_copy(src_hbm.at[pl.ds(r * EPAD + wid * EPW, EPW)], src_v)
        pltpu.sync_copy(dst_hbm.at[pl.ds(r * EPAD + wid * EPW, EPW)], dst_v)

        @pl.loop(0, P)
        def _pass(p):
            base = p * CH

            for j in range(SPS // ZR):
                pltpu.sync_copy(zrows_v, acc_s.at[pl.ds(s * SPS + j * ZR, ZR)])
            pltpu.sync_copy(zdeg_v, dega_s.at[pl.ds(s * SPS, SPS)])
            plsc.subcore_barrier()

            @pl.loop(0, NWAVE)
            def _wave(g):
                eb = g * WV
                for k in range(WV // 16):
                    dv = dst_v[pl.ds(eb + k * 16, 16)]
                    sv = src_v[pl.ds(eb + k * 16, 16)]
                    lanepos = iota16 + k * 16
                    spread = lanepos & 15
                    offv = dv - base
                    inr = (offv >= 0) & (offv < CH)
                    off_v[pl.ds(k * 16, 16)] = jnp.where(inr, offv, lanepos)
                    gsrc_v[pl.ds(k * 16, 16)] = jnp.where(
                        inr, sv, N_NODES + spread)
                    didx_v[pl.ds(k * 16, 16)] = jnp.where(
                        inr, spread, 16 + spread)
                d1 = pltpu.async_copy(x_hbm.at[gsrc_v], rows_v, gsem)
                d2 = pltpu.async_copy(onestab_hbm.at[didx_v], dones_v, g2sem)
                d1.wait()
                d2.wait()
                pltpu.sync_copy(rows_v, acc_s.at[off_v], add=True)
                pltpu.sync_copy(dones_v, dega_s.at[off_v], add=True)

            plsc.subcore_barrier()
            orow = (c * R + r) * NP + base + s * SPS
            pltpu.sync_copy(acc_s.at[pl.ds(s * SPS, SPS)],
                            agg_hbm.at[pl.ds(orow, SPS)])
            pltpu.sync_copy(dega_s.at[pl.ds(s * SPS, SPS)],
                            deg_hbm.at[pl.ds(orow, SPS)])
            plsc.subcore_barrier()


def _sc_segment_sums(src, dst, x, z128, z8, onestab):
    mesh = plsc.VectorSubcoreMesh(core_axis_name="c", subcore_axis_name="s",
                                  num_cores=NC, num_subcores=NS)
    f = pl.kernel(
        _sc_body,
        out_type=(
            jax.ShapeDtypeStruct((NC * R * NP, D), jnp.float32),
            jax.ShapeDtypeStruct((NC * R * NP, 8), jnp.float32),
        ),
        mesh=mesh,
        compiler_params=pltpu.CompilerParams(use_tc_tiling_on_sc=False),
        scratch_types=[
            pltpu.VMEM((EPW,), jnp.int32),
            pltpu.VMEM((EPW,), jnp.int32),
            pltpu.VMEM((WV,), jnp.int32),
            pltpu.VMEM((WV, D), jnp.float32),
            pltpu.VMEM((WV,), jnp.int32),
            pltpu.VMEM((WV,), jnp.int32),
            pltpu.VMEM((WV, 8), jnp.float32),
            pltpu.VMEM((ZR, D), jnp.float32),
            pltpu.VMEM((SPS, 8), jnp.float32),
            pltpu.VMEM_SHARED((CH + WV, D), jnp.float32),
            pltpu.VMEM_SHARED((CH + WV, 8), jnp.float32),
            pltpu.SemaphoreType.DMA,
            pltpu.SemaphoreType.DMA,
        ],
    )
    return f(src, dst, x, z128, z8, onestab)


BLK = 1000


def _tc_body(coeff_ref, agg_ref, deg_ref, x_ref, basis_ref, lw_ref, bias_ref,
             out_ref):
    a4 = agg_ref[...]
    a = a4[0] + a4[1]
    d4 = deg_ref[...]
    d = d4[0, :, :, 0] + d4[1, :, :, 0]
    an = a / jnp.maximum(d, 1.0)[:, :, None]
    u0 = (coeff_ref[0, 0] * an[0] + coeff_ref[1, 0] * an[1]
          + coeff_ref[2, 0] * an[2])
    u1 = (coeff_ref[0, 1] * an[0] + coeff_ref[1, 1] * an[1]
          + coeff_ref[2, 1] * an[2])
    b = basis_ref[...]
    acc = jnp.dot(u0, b[0], preferred_element_type=jnp.float32)
    acc = acc + jnp.dot(u1, b[1], preferred_element_type=jnp.float32)
    acc = acc + jnp.dot(x_ref[...], lw_ref[...],
                        preferred_element_type=jnp.float32)
    out_ref[...] = acc + bias_ref[...]


def _tc_combine(coeff, agg, deg8, x, basis, loop_weight, bias):
    grid = N_NODES // BLK
    return pl.pallas_call(
        _tc_body,
        grid=(grid,),
        in_specs=[
            pl.BlockSpec(memory_space=pltpu.SMEM),
            pl.BlockSpec((NC, R, BLK, D), lambda i: (0, 0, i, 0)),
            pl.BlockSpec((NC, R, BLK, 8), lambda i: (0, 0, i, 0)),
            pl.BlockSpec((BLK, D), lambda i: (i, 0)),
            pl.BlockSpec((NB, D, D), lambda i: (0, 0, 0)),
            pl.BlockSpec((D, D), lambda i: (0, 0)),
            pl.BlockSpec((1, D), lambda i: (0, 0)),
        ],
        out_specs=pl.BlockSpec((BLK, D), lambda i: (i, 0)),
        out_shape=jax.ShapeDtypeStruct((N_NODES, D), jnp.float32),
    )(coeff, agg, deg8, x, basis, loop_weight, bias)


def kernel(x, basis, coeff, loop_weight, bias, edge_index_0, edge_index_1,
           edge_index_2):
    x = x.astype(jnp.float32)
    src = jnp.stack([edge_index_0[0], edge_index_1[0],
                     edge_index_2[0]]).astype(jnp.int32)
    dst = jnp.stack([edge_index_0[1], edge_index_1[1],
                     edge_index_2[1]]).astype(jnp.int32)
    pad = EPAD - N_EDGES
    src = jnp.pad(src, ((0, 0), (0, pad))).reshape(-1)
    dst = jnp.pad(dst, ((0, 0), (0, pad)),
                  constant_values=2 ** 30).reshape(-1)

    z128 = jnp.zeros((ZR, D), jnp.float32)
    z8 = jnp.zeros((SPS, 8), jnp.float32)
    onestab = jnp.concatenate([jnp.ones((16, 8), jnp.float32),
                               jnp.zeros((16, 8), jnp.float32)])
    xp = jnp.concatenate([x, jnp.zeros((16, D), jnp.float32)])

    agg, deg8 = _sc_segment_sums(src, dst, xp, z128, z8, onestab)
    agg = agg.reshape(NC, R, NP, D)
    deg8 = deg8.reshape(NC, R, NP, 8)
    return _tc_combine(coeff, agg, deg8, x, basis, loop_weight,
                       bias.reshape(1, D))


# speedup vs baseline: 7.6939x; 7.6939x over previous
import functools

import jax
import jax.numpy as jnp
from jax import lax
from jax.experimental import pallas as pl
from jax.experimental.pallas import tpu as pltpu
from jax.experimental.pallas import tpu_sc as plsc

N_NODES = 100000
N_EDGES = 200000
D = 128
R = 3
NB = 2

NC = 2
NS = 16
NW = NC * NS

EPW = 6272
EPAD = NW * EPW
G = 128
GPW = EPW // G

CH = 13312
SPS = CH // NS
P = 8
NP = P * CH

ZR = 208


def _sc_body(src_hbm, dst_hbm, x_hbm, z128_hbm, z8_hbm, one8_hbm,
             agg_hbm, deg_hbm,
             src_v, dst_v, off_v, rows_v, ones_v, zrows_v, zdeg_v,
             acc_s, dega_s, sem):
    c = lax.axis_index("c")
    s = lax.axis_index("s")
    wid = s * NC + c

    pltpu.sync_copy(z128_hbm, zrows_v)
    pltpu.sync_copy(z8_hbm, zdeg_v)
    pltpu.sync_copy(one8_hbm, ones_v)

    for r in range(R):
        pltpu.sync_copy(src_hbm.at[pl.ds(r * EPAD + wid * EPW, EPW)], src_v)
        pltpu.sync_copy(dst_hbm.at[pl.ds(r * EPAD + wid * EPW, EPW)], dst_v)

        @pl.loop(0, P)
        def _pass(p):
            base = p * CH

            for j in range(SPS // ZR):
                pltpu.sync_copy(zrows_v, acc_s.at[pl.ds(s * SPS + j * ZR, ZR)])
            pltpu.sync_copy(zdeg_v, dega_s.at[pl.ds(s * SPS, SPS)])
            plsc.subcore_barrier()

            @pl.loop(0, GPW)
            def _chunk(g):
                eb = g * G
                for k in range(G // 16):
                    dv = dst_v[pl.ds(eb + k * 16, 16)]
                    offv = dv - base
                    inr = (offv >= 0) & (offv < CH)
                    off_v[pl.ds(k * 16, 16)] = jnp.where(inr, offv, CH)
                pltpu.async_copy(
                    x_hbm.at[src_v.at[pl.ds(eb, G)]], rows_v, sem).wait()
                pltpu.sync_copy(rows_v, acc_s.at[off_v], add=True)
                pltpu.sync_copy(ones_v, dega_s.at[off_v], add=True)

            plsc.subcore_barrier()
            orow = (c * R + r) * NP + base + s * SPS
            pltpu.sync_copy(acc_s.at[pl.ds(s * SPS, SPS)],
                            agg_hbm.at[pl.ds(orow, SPS)])
            pltpu.sync_copy(dega_s.at[pl.ds(s * SPS, SPS)],
                            deg_hbm.at[pl.ds(orow, SPS)])
            plsc.subcore_barrier()


def _sc_segment_sums(src, dst, x, z128, z8, one8):
    mesh = plsc.VectorSubcoreMesh(core_axis_name="c", subcore_axis_name="s",
                                  num_cores=NC, num_subcores=NS)
    f = pl.kernel(
        _sc_body,
        out_type=(
            jax.ShapeDtypeStruct((NC * R * NP, D), jnp.bfloat16),
            jax.ShapeDtypeStruct((NC * R * NP, 8), jnp.float32),
        ),
        mesh=mesh,
        compiler_params=pltpu.CompilerParams(use_tc_tiling_on_sc=False),
        scratch_types=[
            pltpu.VMEM((EPW,), jnp.int32),
            pltpu.VMEM((EPW,), jnp.int32),
            pltpu.VMEM((G,), jnp.int32),
            pltpu.VMEM((G, D), jnp.bfloat16),
            pltpu.VMEM((G, 8), jnp.float32),
            pltpu.VMEM((ZR, D), jnp.bfloat16),
            pltpu.VMEM((SPS, 8), jnp.float32),
            pltpu.VMEM_SHARED((CH + 8, D), jnp.bfloat16),
            pltpu.VMEM_SHARED((CH + 8, 8), jnp.float32),
            pltpu.SemaphoreType.DMA,
        ],
    )
    return f(src, dst, x, z128, z8, one8)


BLK = 1000


def _tc_body(coeff_ref, agg_ref, deg_ref, x_ref, basis_ref, lw_ref, bias_ref,
             out_ref):
    a4 = agg_ref[...].astype(jnp.float32)
    a = a4[0] + a4[1]
    d4 = deg_ref[...]
    d = d4[0, :, :, 0] + d4[1, :, :, 0]
    an = a / jnp.maximum(d, 1.0)[:, :, None]
    u0 = (coeff_ref[0, 0] * an[0] + coeff_ref[1, 0] * an[1]
          + coeff_ref[2, 0] * an[2])
    u1 = (coeff_ref[0, 1] * an[0] + coeff_ref[1, 1] * an[1]
          + coeff_ref[2, 1] * an[2])
    b = basis_ref[...]
    acc = jnp.dot(u0, b[0], preferred_element_type=jnp.float32)
    acc = acc + jnp.dot(u1, b[1], preferred_element_type=jnp.float32)
    acc = acc + jnp.dot(x_ref[...], lw_ref[...],
                        preferred_element_type=jnp.float32)
    out_ref[...] = acc + bias_ref[...]


def _tc_combine(coeff, agg, deg8, x, basis, loop_weight, bias):
    grid = N_NODES // BLK
    return pl.pallas_call(
        _tc_body,
        grid=(grid,),
        in_specs=[
            pl.BlockSpec(memory_space=pltpu.SMEM),
            pl.BlockSpec((NC, R, BLK, D), lambda i: (0, 0, i, 0)),
            pl.BlockSpec((NC, R, BLK, 8), lambda i: (0, 0, i, 0)),
            pl.BlockSpec((BLK, D), lambda i: (i, 0)),
            pl.BlockSpec((NB, D, D), lambda i: (0, 0, 0)),
            pl.BlockSpec((D, D), lambda i: (0, 0)),
            pl.BlockSpec((1, D), lambda i: (0, 0)),
        ],
        out_specs=pl.BlockSpec((BLK, D), lambda i: (i, 0)),
        out_shape=jax.ShapeDtypeStruct((N_NODES, D), jnp.float32),
    )(coeff, agg, deg8, x, basis, loop_weight, bias)


def kernel(x, basis, coeff, loop_weight, bias, edge_index_0, edge_index_1,
           edge_index_2):
    x = x.astype(jnp.float32)
    src = jnp.stack([edge_index_0[0], edge_index_1[0],
                     edge_index_2[0]]).astype(jnp.int32)
    dst = jnp.stack([edge_index_0[1], edge_index_1[1],
                     edge_index_2[1]]).astype(jnp.int32)
    pad = EPAD - N_EDGES
    src = jnp.pad(src, ((0, 0), (0, pad))).reshape(-1)
    dst = jnp.pad(dst, ((0, 0), (0, pad)),
                  constant_values=2 ** 30).reshape(-1)

    z128 = jnp.zeros((ZR, D), jnp.bfloat16)
    z8 = jnp.zeros((SPS, 8), jnp.float32)
    one8 = jnp.ones((G, 8), jnp.float32)

    agg, deg8 = _sc_segment_sums(src, dst, x.astype(jnp.bfloat16), z128, z8, one8)
    agg = agg.reshape(NC, R, NP, D)
    deg8 = deg8.reshape(NC, R, NP, 8)
    return _tc_combine(coeff, agg, deg8, x, basis, loop_weight,
                       bias.reshape(1, D))
